# Initial kernel scaffold; baseline (speedup 1.0000x reference)
import jax
import jax.numpy as jnp
from jax.experimental import pallas as pl


def kernel(input_ids, emb_table, lin_weight):
    raise NotImplementedError("write your pallas kernel here")


# speedup vs baseline: 4.3746x; 4.3746x over previous
import functools

import jax
import jax.numpy as jnp
from jax import lax
from jax.experimental import pallas as pl
from jax.experimental.pallas import tpu as pltpu
from jax.experimental.pallas import tpu_sc as plsc


def _transform_table(emb_table, lin_weight, blk=1000):
    v, w = emb_table.shape

    def body(emb_ref, w_ref, out_ref):
        out_ref[...] = lax.dot_general(
            emb_ref[...], w_ref[...],
            (((1,), (1,)), ((), ())),
            preferred_element_type=jnp.float32,
        )

    return pl.pallas_call(
        body,
        grid=(v // blk,),
        in_specs=[
            pl.BlockSpec((blk, w), lambda i: (i, 0)),
            pl.BlockSpec((w, w), lambda i: (0, 0)),
        ],
        out_specs=pl.BlockSpec((blk, w), lambda i: (i, 0)),
        out_shape=jax.ShapeDtypeStruct((v, w), jnp.float32),
    )(emb_table, lin_weight)


def _make_gather(v, w, ntok, nc, ns, chunk):
    nw = nc * ns
    per_w = ntok // nw
    n_chunks = per_w // chunk
    lanes = 16
    mesh = plsc.VectorSubcoreMesh(core_axis_name="c", subcore_axis_name="s")

    @functools.partial(
        pl.kernel,
        out_type=(
            jax.ShapeDtypeStruct((ntok, w), jnp.float32),
            jax.ShapeDtypeStruct((nw, lanes), jnp.float32),
        ),
        mesh=mesh,
        scratch_types=[
            pltpu.VMEM((n_chunks, chunk), jnp.int32),
            pltpu.VMEM((chunk, w), jnp.float32),
            pltpu.VMEM((lanes,), jnp.float32),
            pltpu.SemaphoreType.DMA,
        ],
    )
    def gather(idx_hbm, table_hbm, out_hbm, psum_hbm, idx_v, rows_v, acc_v, sem):
        wid = lax.axis_index("s") * nc + lax.axis_index("c")
        base = wid * per_w
        pltpu.sync_copy(idx_hbm.at[wid], idx_v)
        acc_v[...] = jnp.zeros((lanes,), jnp.float32)

        def chunk_body(j, carry):
            pltpu.async_copy(table_hbm.at[idx_v.at[j]], rows_v, sem).wait()
            pltpu.sync_copy(rows_v, out_hbm.at[pl.ds(base + j * chunk, chunk)])

            def row_body(r, acc):
                for cpos in range(w // lanes):
                    acc = acc + rows_v[r, pl.ds(cpos * lanes, lanes)]
                return acc

            acc_v[...] = lax.fori_loop(0, chunk, row_body, acc_v[...])
            return carry

        lax.fori_loop(0, n_chunks, chunk_body, 0)
        pltpu.sync_copy(acc_v, psum_hbm.at[wid])

    return gather


def kernel(input_ids, emb_table, lin_weight):
    b, s = input_ids.shape
    v, w = emb_table.shape
    ntok = b * s

    info = plsc.get_sparse_core_info()
    nc, ns = info.num_cores, info.num_subcores
    chunk = 128
    nw = nc * ns

    transformed = _transform_table(emb_table, lin_weight)
    idx = input_ids.reshape(nw, ntok // (nw * chunk), chunk)
    out_flat, psum = _make_gather(v, w, ntok, nc, ns, chunk)(idx, transformed)
    outputs = out_flat.reshape(b, s, w)
    loss = jnp.sum(psum) / (ntok * w)
    return outputs, loss


# speedup vs baseline: 7.0268x; 1.6063x over previous
import functools

import jax
import jax.numpy as jnp
from jax import lax
from jax.experimental import pallas as pl
from jax.experimental.pallas import tpu as pltpu
from jax.experimental.pallas import tpu_sc as plsc


def _transform_table(emb_table, lin_weight, blk=1000):
    v, w = emb_table.shape

    def body(emb_ref, w_ref, out_ref):
        out_ref[...] = lax.dot_general(
            emb_ref[...], w_ref[...],
            (((1,), (1,)), ((), ())),
            preferred_element_type=jnp.float32,
        )

    return pl.pallas_call(
        body,
        grid=(v // blk,),
        in_specs=[
            pl.BlockSpec((blk, w), lambda i: (i, 0)),
            pl.BlockSpec((w, w), lambda i: (0, 0)),
        ],
        out_specs=pl.BlockSpec((blk, w), lambda i: (i, 0)),
        out_shape=jax.ShapeDtypeStruct((v, w), jnp.float32),
    )(emb_table, lin_weight)


def _make_gather(v, w, ntok, nc, ns, chunk, nbuf=4):
    nw = nc * ns
    per_w = ntok // nw
    n_chunks = per_w // chunk
    lanes = 16
    mesh = plsc.VectorSubcoreMesh(core_axis_name="c", subcore_axis_name="s")

    @functools.partial(
        pl.kernel,
        out_type=(
            jax.ShapeDtypeStruct((ntok, w), jnp.float32),
            jax.ShapeDtypeStruct((nw, lanes), jnp.float32),
        ),
        mesh=mesh,
        scratch_types=(
            [pltpu.VMEM((n_chunks, chunk), jnp.int32)]
            + [pltpu.VMEM((chunk, w), jnp.float32) for _ in range(nbuf)]
            + [pltpu.VMEM((lanes,), jnp.float32)]
            + [pltpu.SemaphoreType.DMA for _ in range(2 * nbuf)]
        ),
    )
    def gather(idx_hbm, table_hbm, out_hbm, psum_hbm, idx_v, *scr):
        bufs = scr[:nbuf]
        acc_v = scr[nbuf]
        gsems = scr[nbuf + 1:nbuf + 1 + nbuf]
        osems = scr[nbuf + 1 + nbuf:]
        wid = lax.axis_index("s") * nc + lax.axis_index("c")
        base = wid * per_w
        pltpu.sync_copy(idx_hbm.at[wid], idx_v)
        acc_v[...] = jnp.zeros((lanes,), jnp.float32)
        for b in range(nbuf):
            pltpu.async_copy(table_hbm.at[idx_v.at[b]], bufs[b], gsems[b])

        def sum_buf(buf, acc):
            def row_body(r, a):
                for cpos in range(w // lanes):
                    a = a + buf[r, pl.ds(cpos * lanes, lanes)]
                return a
            return lax.fori_loop(0, chunk, row_body, acc)

        def out_slice(j):
            return out_hbm.at[pl.ds(base + j * chunk, chunk)]

        def slot(j, b, issue_next):
            pltpu.make_async_copy(table_hbm.at[idx_v.at[j]], bufs[b], gsems[b]).wait()
            acc_v[...] = sum_buf(bufs[b], acc_v[...])
            pltpu.async_copy(bufs[b], out_slice(j), osems[b])
            if issue_next:
                pltpu.make_async_copy(bufs[b], out_slice(j), osems[b]).wait()
                pltpu.async_copy(table_hbm.at[idx_v.at[j + nbuf]], bufs[b], gsems[b])

        def body(it, carry):
            j0 = it * nbuf
            for b in range(nbuf):
                slot(j0 + b, b, True)
            return carry

        lax.fori_loop(0, n_chunks // nbuf - 1, body, 0)
        for b in range(nbuf):
            j = n_chunks - nbuf + b
            slot(j, b, False)
            pltpu.make_async_copy(bufs[b], out_slice(j), osems[b]).wait()
        pltpu.sync_copy(acc_v, psum_hbm.at[wid])

    return gather


def kernel(input_ids, emb_table, lin_weight):
    b, s = input_ids.shape
    v, w = emb_table.shape
    ntok = b * s

    info = plsc.get_sparse_core_info()
    nc, ns = info.num_cores, info.num_subcores
    chunk = 128
    nw = nc * ns

    transformed = _transform_table(emb_table, lin_weight)
    idx = input_ids.reshape(nw, ntok // (nw * chunk), chunk)
    out_flat, psum = _make_gather(v, w, ntok, nc, ns, chunk)(idx, transformed)
    outputs = out_flat.reshape(b, s, w)
    loss = jnp.sum(psum) / (ntok * w)
    return outputs, loss


# speedup vs baseline: 8.2218x; 1.1701x over previous
import functools

import jax
import jax.numpy as jnp
from jax import lax
from jax.experimental import pallas as pl
from jax.experimental.pallas import tpu as pltpu
from jax.experimental.pallas import tpu_sc as plsc


def _transform_table(emb_table, lin_weight, blk=1000):
    v, w = emb_table.shape

    def body(emb_ref, w_ref, out_ref):
        out_ref[...] = lax.dot_general(
            emb_ref[...], w_ref[...],
            (((1,), (1,)), ((), ())),
            preferred_element_type=jnp.float32,
        )

    return pl.pallas_call(
        body,
        grid=(v // blk,),
        in_specs=[
            pl.BlockSpec((blk, w), lambda i: (i, 0)),
            pl.BlockSpec((w, w), lambda i: (0, 0)),
        ],
        out_specs=pl.BlockSpec((blk, w), lambda i: (i, 0)),
        out_shape=jax.ShapeDtypeStruct((v, w), jnp.float32),
    )(emb_table, lin_weight)


def _make_gather(v, w, ntok, nc, ns, chunk, nbuf=4):
    nw = nc * ns
    per_w = ntok // nw
    n_chunks = per_w // chunk
    lanes = 16
    mesh = plsc.VectorSubcoreMesh(core_axis_name="c", subcore_axis_name="s")

    @functools.partial(
        pl.kernel,
        out_type=(
            jax.ShapeDtypeStruct((ntok, w), jnp.float32),
            jax.ShapeDtypeStruct((nw, lanes), jnp.float32),
        ),
        mesh=mesh,
        scratch_types=(
            [pltpu.VMEM((n_chunks, chunk), jnp.int32)]
            + [pltpu.VMEM((chunk, w), jnp.float32) for _ in range(nbuf)]
            + [pltpu.VMEM((lanes,), jnp.float32)]
            + [pltpu.SemaphoreType.DMA for _ in range(2 * nbuf)]
        ),
    )
    def gather(idx_hbm, table_hbm, out_hbm, psum_hbm, idx_v, *scr):
        bufs = scr[:nbuf]
        acc_v = scr[nbuf]
        gsems = scr[nbuf + 1:nbuf + 1 + nbuf]
        osems = scr[nbuf + 1 + nbuf:]
        wid = lax.axis_index("s") * nc + lax.axis_index("c")
        base = wid * per_w
        pltpu.sync_copy(idx_hbm.at[wid], idx_v)
        acc_v[...] = jnp.zeros((lanes,), jnp.float32)
        for b in range(nbuf):
            pltpu.async_copy(table_hbm.at[idx_v.at[b]], bufs[b], gsems[b])

        def sum_buf(buf, acc):
            def row_body(r, a):
                for cpos in range(w // lanes):
                    a = a + buf[r, pl.ds(cpos * lanes, lanes)]
                return a
            return lax.fori_loop(0, chunk, row_body, acc)

        def out_slice(j):
            return out_hbm.at[pl.ds(base + j * chunk, chunk)]

        def slot(j, b, issue_next):
            pltpu.make_async_copy(table_hbm.at[idx_v.at[j]], bufs[b], gsems[b]).wait()
            pltpu.async_copy(bufs[b], out_slice(j), osems[b])
            acc_v[...] = sum_buf(bufs[b], acc_v[...])
            if issue_next:
                pltpu.make_async_copy(bufs[b], out_slice(j), osems[b]).wait()
                pltpu.async_copy(table_hbm.at[idx_v.at[j + nbuf]], bufs[b], gsems[b])

        def body(it, carry):
            j0 = it * nbuf
            for b in range(nbuf):
                slot(j0 + b, b, True)
            return carry

        lax.fori_loop(0, n_chunks // nbuf - 1, body, 0)
        for b in range(nbuf):
            j = n_chunks - nbuf + b
            slot(j, b, False)
            pltpu.make_async_copy(bufs[b], out_slice(j), osems[b]).wait()
        pltpu.sync_copy(acc_v, psum_hbm.at[wid])

    return gather


def kernel(input_ids, emb_table, lin_weight):
    b, s = input_ids.shape
    v, w = emb_table.shape
    ntok = b * s

    info = plsc.get_sparse_core_info()
    nc, ns = info.num_cores, info.num_subcores
    chunk = 128
    nw = nc * ns

    transformed = _transform_table(emb_table, lin_weight)
    idx = input_ids.reshape(nw, ntok // (nw * chunk), chunk)
    out_flat, psum = _make_gather(v, w, ntok, nc, ns, chunk)(idx, transformed)
    outputs = out_flat.reshape(b, s, w)
    loss = jnp.sum(psum) / (ntok * w)
    return outputs, loss
